# baseline (speedup 1.0000x reference)
import functools

import jax
import jax.numpy as jnp
from jax import lax
from jax.experimental import pallas as pl
from jax.experimental.pallas import tpu as pltpu
from jax.experimental.pallas import tpu_sc as plsc

N = 10000
E = 320000
D = 128
H = 128
L = 2
K = 5

BN = 400
NBLK = N // BN


def _gates_body(tx_ref, tpad_ref, wcat_ref, whz_ref, whr_ref, bz_ref, br_ref,
                bh_ref, z_ref, rsplit_ref, ahx_ref):
    acc = jnp.zeros((BN, 3 * H), jnp.float32)
    for k in range(K):
        for h in range(2):
            acc += jnp.dot(tx_ref[k, h], wcat_ref[k, 64 * h:64 * h + 64, :],
                           preferred_element_type=jnp.float32)
    tblk = tpad_ref[:, :K]
    czs = jnp.sum(whz_ref[...], axis=1)
    crs = jnp.sum(whr_ref[...], axis=1)
    az = acc[:, :H] + jnp.dot(tblk, czs, preferred_element_type=jnp.float32) \
        + bz_ref[0]
    ar = acc[:, H:2 * H] + jnp.dot(tblk, crs, preferred_element_type=jnp.float32) \
        + br_ref[0]
    ah = acc[:, 2 * H:] + bh_ref[0]
    z = jax.nn.sigmoid(az)
    r = jax.nn.sigmoid(ar)
    z_ref[...] = z
    rsplit_ref[0] = r[:, :64]
    rsplit_ref[1] = r[:, 64:]
    ahx_ref[...] = ah


def _gates_call(tx_all, tpad, wcat, whz, whr, bz, br, bh):
    return pl.pallas_call(
        _gates_body,
        grid=(NBLK,),
        in_specs=[
            pl.BlockSpec((K, 2, BN, 64), lambda i: (0, 0, i, 0)),
            pl.BlockSpec((BN, 8), lambda i: (i, 0)),
            pl.BlockSpec((K, D, 3 * H), lambda i: (0, 0, 0)),
            pl.BlockSpec((K, D, H), lambda i: (0, 0, 0)),
            pl.BlockSpec((K, D, H), lambda i: (0, 0, 0)),
            pl.BlockSpec((1, H), lambda i: (0, 0)),
            pl.BlockSpec((1, H), lambda i: (0, 0)),
            pl.BlockSpec((1, H), lambda i: (0, 0)),
        ],
        out_specs=[
            pl.BlockSpec((BN, H), lambda i: (i, 0)),
            pl.BlockSpec((2, BN, 64), lambda i: (0, i, 0)),
            pl.BlockSpec((BN, H), lambda i: (i, 0)),
        ],
        out_shape=[
            jax.ShapeDtypeStruct((N, H), jnp.float32),
            jax.ShapeDtypeStruct((2, N, 64), jnp.float32),
            jax.ShapeDtypeStruct((N, H), jnp.float32),
        ],
    )(tx_all, tpad, wcat, whz, whr, bz, br, bh)


def _out_body(tr_ref, whh_ref, ahx_ref, z_ref, bh_ref, h_ref, hsplit_ref):
    acc = ahx_ref[...] + bh_ref[0]
    for k in range(K):
        for h in range(2):
            acc += jnp.dot(tr_ref[k, h], whh_ref[k, 64 * h:64 * h + 64, :],
                           preferred_element_type=jnp.float32)
    ht = jnp.tanh(acc)
    z = z_ref[...]
    out = z + (1.0 - z) * ht
    hv = jnp.maximum(out, 0.0)
    h_ref[...] = hv
    hsplit_ref[0] = hv[:, :64]
    hsplit_ref[1] = hv[:, 64:]


def _out_call(tr_all, whh, ahx, z, bh):
    return pl.pallas_call(
        _out_body,
        grid=(NBLK,),
        in_specs=[
            pl.BlockSpec((K, 2, BN, 64), lambda i: (0, 0, i, 0)),
            pl.BlockSpec((K, D, H), lambda i: (0, 0, 0)),
            pl.BlockSpec((BN, H), lambda i: (i, 0)),
            pl.BlockSpec((BN, H), lambda i: (i, 0)),
            pl.BlockSpec((1, H), lambda i: (0, 0)),
        ],
        out_specs=[
            pl.BlockSpec((BN, H), lambda i: (i, 0)),
            pl.BlockSpec((2, BN, 64), lambda i: (0, i, 0)),
        ],
        out_shape=[
            jax.ShapeDtypeStruct((N, H), jnp.float32),
            jax.ShapeDtypeStruct((2, N, 64), jnp.float32),
        ],
    )(tr_all, whh, ahx, z, bh)


NCH = 160
EPT = NCH * 128
EPAD = EPT * 16
NROW = 10240
NPT = NROW // 16
NPAIR = NCH // 2

_sc_mesh = plsc.VectorSubcoreMesh(core_axis_name="c", subcore_axis_name="s")


def _series_sc_body(x2, esrc, edst, enorm, t1, t2, t3, t4,
                    acc, srcv, dstv, normv, rb0, rb1, gsem0, gsem1):
    c = lax.axis_index("c")
    s = lax.axis_index("s")
    cN = c * NROW
    zv = jnp.zeros((16,), jnp.float32)

    pltpu.sync_copy(esrc.at[s], srcv)
    pltpu.sync_copy(edst.at[s], dstv)
    pltpu.sync_copy(enorm.at[s], normv)

    def _offset_row(j, _):
        for q in range(8):
            sl = pl.ds(q * 16, 16)
            srcv[j, sl] = srcv[j, sl] + cN
        return 0
    lax.fori_loop(0, NCH, _offset_row, 0)

    def _zero_rb(rb):
        def _z(r, _):
            for q in range(4):
                rb[r, pl.ds(q * 16, 16)] = zv
            return 0
        lax.fori_loop(0, 128, _z, 0)

    _zero_rb(rb0)
    for blk in range(NPT // 128):
        pltpu.sync_copy(rb0, acc.at[pl.ds(s * NPT + blk * 128, 128)])
    plsc.subcore_barrier()

    def _scale(rb, j):
        def _g(g, _):
            nv = normv[j, pl.ds(g * 16, 16)]
            for i in range(16):
                sc = nv[i]
                e = g * 16 + i
                for q in range(4):
                    sl = pl.ds(q * 16, 16)
                    rb[e, sl] = rb[e, sl] * sc
            return 0
        lax.fori_loop(0, 8, _g, 0)

    def _hop(src_tab, prev_tab, out_tab, rezero):
        pltpu.async_copy(src_tab.at[srcv.at[0]], rb0, gsem0)

        def _pair(jj, _):
            j0 = 2 * jj
            j1 = j0 + 1
            pltpu.make_async_copy(src_tab.at[pl.ds(0, 128)], rb0, gsem0).wait()
            pltpu.async_copy(src_tab.at[srcv.at[j1]], rb1, gsem1)
            _scale(rb0, j0)
            pltpu.sync_copy(rb0, acc.at[dstv.at[j0]], add=True)
            pltpu.make_async_copy(src_tab.at[pl.ds(0, 128)], rb1, gsem1).wait()

            @pl.when(jj + 1 < NPAIR)
            def _():
                pltpu.async_copy(src_tab.at[srcv.at[j0 + 2]], rb0, gsem0)

            _scale(rb1, j1)
            pltpu.sync_copy(rb1, acc.at[dstv.at[j1]], add=True)
            return 0
        lax.fori_loop(0, NPAIR, _pair, 0)
        plsc.subcore_barrier()

        base = s * NPT
        obase = cN + s * NPT
        for blk in range(NPT // 128):
            pltpu.sync_copy(acc.at[pl.ds(base + blk * 128, 128)], rb0)
            if prev_tab is not None:
                pltpu.sync_copy(prev_tab.at[pl.ds(obase + blk * 128, 128)],
                                rb1)

                def _c(r, _):
                    for q in range(4):
                        sl = pl.ds(q * 16, 16)
                        rb0[r, sl] = 2.0 * rb0[r, sl] - rb1[r, sl]
                    return 0
                lax.fori_loop(0, 128, _c, 0)
            pltpu.sync_copy(rb0, out_tab.at[pl.ds(obase + blk * 128, 128)])
        if rezero:
            _zero_rb(rb0)
            for blk in range(NPT // 128):
                pltpu.sync_copy(rb0, acc.at[pl.ds(base + blk * 128, 128)])
        plsc.subcore_barrier()

    _hop(x2, None, t1, True)
    _hop(t1, x2, t2, True)
    _hop(t2, t1, t3, True)
    _hop(t3, t2, t4, False)


_series_sc_call = pl.kernel(
    _series_sc_body,
    out_type=[jax.ShapeDtypeStruct((2 * NROW, 64), jnp.float32)] * 4,
    mesh=_sc_mesh,
    scratch_types=[
        pltpu.VMEM_SHARED((NROW, 64), jnp.float32),
        pltpu.VMEM((NCH, 128), jnp.int32),
        pltpu.VMEM((NCH, 128), jnp.int32),
        pltpu.VMEM((NCH, 128), jnp.float32),
        pltpu.VMEM((128, 64), jnp.float32),
        pltpu.VMEM((128, 64), jnp.float32),
        pltpu.SemaphoreType.DMA,
        pltpu.SemaphoreType.DMA,
    ],
    compiler_params=pltpu.CompilerParams(use_tc_tiling_on_sc=False),
)


def _series_sc(x2, esrc, edst, enorm):
    return _series_sc_call(x2, esrc, edst, enorm)


def kernel(edge_index, edge_weight, emb, weights, biases):
    src, dst = edge_index[0], edge_index[1]

    npad = EPAD - E
    esrc = jnp.concatenate([src, jnp.zeros((npad,), jnp.int32)]) \
        .reshape(16, NCH, 128)
    edst = jnp.concatenate(
        [dst, N + (jnp.arange(npad, dtype=jnp.int32) % 16)]) \
        .reshape(16, NCH, 128)

    deg = jax.ops.segment_sum(edge_weight, src, num_segments=N)
    safe_deg = jnp.where(deg > 0, deg, 1.0)
    dis = jnp.where(deg > 0, lax.rsqrt(safe_deg), 0.0)
    norm = -dis[src] * edge_weight * dis[dst]
    enorm = jnp.concatenate([norm, jnp.zeros((npad,), jnp.float32)]) \
        .reshape(16, NCH, 128)

    def prop1(v):
        return jax.ops.segment_sum(norm * v[src], dst, num_segments=N)

    t1 = prop1(jnp.ones((N,), jnp.float32))
    t2 = 2.0 * prop1(t1) - 1.0
    t3 = 2.0 * prop1(t2) - t1
    t4 = 2.0 * prop1(t3) - t2
    tpad = jnp.concatenate(
        [jnp.ones((N, 1), jnp.float32),
         jnp.stack([t1, t2, t3, t4], axis=1),
         jnp.zeros((N, 3), jnp.float32)], axis=1)

    def series_all(xsplit):
        x2 = jnp.pad(xsplit, ((0, 0), (0, NROW - N), (0, 0))) \
            .reshape(2 * NROW, 64)
        ts4 = _series_sc(x2, esrc, edst, enorm)
        return jnp.stack(
            [xsplit] + [t.reshape(2, NROW, 64)[:, :N, :] for t in ts4])

    cur = jnp.stack([emb[:, :64], emb[:, 64:]])
    hs = []
    for l in range(L):
        W = weights[l]
        b = biases[l]
        wcat = jnp.concatenate([W[0], W[2], W[4]], axis=-1)
        bz = (b[0] + b[1]).reshape(1, H)
        br = (b[2] + b[3]).reshape(1, H)
        bh4 = b[4].reshape(1, H)
        bh5 = b[5].reshape(1, H)

        tx_all = series_all(cur)
        z, rsplit, ahx = _gates_call(tx_all, tpad, wcat, W[1], W[3],
                                     bz, br, bh4)
        tr_all = series_all(rsplit)
        h, hsplit = _out_call(tr_all, W[5], ahx, z, bh5)
        hs.append(h)
        cur = hsplit
    return (hs[-1], hs[0], hs[1])


# speedup vs baseline: 1.9758x; 1.9758x over previous
import jax
import jax.numpy as jnp
from jax import lax
from jax.experimental import pallas as pl
from jax.experimental.pallas import tpu as pltpu
from jax.experimental.pallas import tpu_sc as plsc

N = 10000
E = 320000
D = 128
H = 128
L = 2
K = 5

BN = 400
NBLK = N // BN


def _gates_body(tx_ref, tpad_ref, wcat_ref, whz_ref, whr_ref, bz_ref, br_ref,
                bh_ref, z_ref, rsplit_ref, ahx_ref):
    acc = jnp.zeros((BN, 3 * H), jnp.float32)
    for k in range(K):
        for h in range(2):
            acc += jnp.dot(tx_ref[k, h], wcat_ref[k, 64 * h:64 * h + 64, :],
                           preferred_element_type=jnp.float32)
    tblk = tpad_ref[:, :K]
    czs = jnp.sum(whz_ref[...], axis=1)
    crs = jnp.sum(whr_ref[...], axis=1)
    az = acc[:, :H] + jnp.dot(tblk, czs, preferred_element_type=jnp.float32) \
        + bz_ref[0]
    ar = acc[:, H:2 * H] + jnp.dot(tblk, crs, preferred_element_type=jnp.float32) \
        + br_ref[0]
    ah = acc[:, 2 * H:] + bh_ref[0]
    z = jax.nn.sigmoid(az)
    r = jax.nn.sigmoid(ar)
    z_ref[...] = z
    rsplit_ref[0] = r[:, :64]
    rsplit_ref[1] = r[:, 64:]
    ahx_ref[...] = ah


def _gates_call(tx_all, tpad, wcat, whz, whr, bz, br, bh):
    return pl.pallas_call(
        _gates_body,
        grid=(NBLK,),
        in_specs=[
            pl.BlockSpec((K, 2, BN, 64), lambda i: (0, 0, i, 0)),
            pl.BlockSpec((BN, 8), lambda i: (i, 0)),
            pl.BlockSpec((K, D, 3 * H), lambda i: (0, 0, 0)),
            pl.BlockSpec((K, D, H), lambda i: (0, 0, 0)),
            pl.BlockSpec((K, D, H), lambda i: (0, 0, 0)),
            pl.BlockSpec((1, H), lambda i: (0, 0)),
            pl.BlockSpec((1, H), lambda i: (0, 0)),
            pl.BlockSpec((1, H), lambda i: (0, 0)),
        ],
        out_specs=[
            pl.BlockSpec((BN, H), lambda i: (i, 0)),
            pl.BlockSpec((2, BN, 64), lambda i: (0, i, 0)),
            pl.BlockSpec((BN, H), lambda i: (i, 0)),
        ],
        out_shape=[
            jax.ShapeDtypeStruct((N, H), jnp.float32),
            jax.ShapeDtypeStruct((2, N, 64), jnp.float32),
            jax.ShapeDtypeStruct((N, H), jnp.float32),
        ],
    )(tx_all, tpad, wcat, whz, whr, bz, br, bh)


def _out_body(tr_ref, whh_ref, ahx_ref, z_ref, bh_ref, h_ref, hsplit_ref):
    acc = ahx_ref[...] + bh_ref[0]
    for k in range(K):
        for h in range(2):
            acc += jnp.dot(tr_ref[k, h], whh_ref[k, 64 * h:64 * h + 64, :],
                           preferred_element_type=jnp.float32)
    ht = jnp.tanh(acc)
    z = z_ref[...]
    out = z + (1.0 - z) * ht
    hv = jnp.maximum(out, 0.0)
    h_ref[...] = hv
    hsplit_ref[0] = hv[:, :64]
    hsplit_ref[1] = hv[:, 64:]


def _out_call(tr_all, whh, ahx, z, bh):
    return pl.pallas_call(
        _out_body,
        grid=(NBLK,),
        in_specs=[
            pl.BlockSpec((K, 2, BN, 64), lambda i: (0, 0, i, 0)),
            pl.BlockSpec((K, D, H), lambda i: (0, 0, 0)),
            pl.BlockSpec((BN, H), lambda i: (i, 0)),
            pl.BlockSpec((BN, H), lambda i: (i, 0)),
            pl.BlockSpec((1, H), lambda i: (0, 0)),
        ],
        out_specs=[
            pl.BlockSpec((BN, H), lambda i: (i, 0)),
            pl.BlockSpec((2, BN, 64), lambda i: (0, i, 0)),
        ],
        out_shape=[
            jax.ShapeDtypeStruct((N, H), jnp.float32),
            jax.ShapeDtypeStruct((2, N, 64), jnp.float32),
        ],
    )(tr_all, whh, ahx, z, bh)


NCH = 160
EPT = NCH * 128
EPAD = EPT * 16
NROW = 10240
NPT = NROW // 16
NPAIR = NCH // 2

_sc_mesh = plsc.VectorSubcoreMesh(core_axis_name="c", subcore_axis_name="s")


def _make_series_body(W):
    NQ = W // 16

    def _series_sc_body(x2, u0, esrc, edst, ew, dis,
                        t1, t2, t3, t4, u1o, u2o, u3o,
                        acc, srcv, dstv, wv, rb0, rb1, gsem0, gsem1, disb):
        c = lax.axis_index("c")
        s = lax.axis_index("s")
        cN = c * NROW
        zv = jnp.zeros((16,), jnp.float32)

        pltpu.sync_copy(esrc.at[s], srcv)
        pltpu.sync_copy(edst.at[s], dstv)
        pltpu.sync_copy(ew.at[s], wv)
        pltpu.sync_copy(dis.at[pl.ds(s * NPT, NPT)], disb)

        def _offset_row(j, _):
            for q in range(8):
                sl = pl.ds(q * 16, 16)
                srcv[j, sl] = srcv[j, sl] + cN
            return 0
        lax.fori_loop(0, NCH, _offset_row, 0)

        def _zero_rb(rb):
            def _z(r, _):
                for q in range(NQ):
                    rb[r, pl.ds(q * 16, 16)] = zv
                return 0
            lax.fori_loop(0, 128, _z, 0)

        _zero_rb(rb0)
        for blk in range(NPT // 128):
            pltpu.sync_copy(rb0, acc.at[pl.ds(s * NPT + blk * 128, 128)])
        plsc.subcore_barrier()

        def _scale(rb, j):
            def _g(g, _):
                nv = wv[j, pl.ds(g * 16, 16)]
                for i in range(16):
                    sc = nv[i]
                    e = g * 16 + i
                    for q in range(NQ):
                        sl = pl.ds(q * 16, 16)
                        rb[e, sl] = rb[e, sl] * sc
                return 0
            lax.fori_loop(0, 8, _g, 0)

        def _hop(src_tab, prev_tab, out_tab, uout_tab):
            pltpu.async_copy(src_tab.at[srcv.at[0]], rb0, gsem0)

            def _pair(jj, _):
                j0 = 2 * jj
                j1 = j0 + 1
                pltpu.make_async_copy(src_tab.at[pl.ds(0, 128)], rb0,
                                      gsem0).wait()
                pltpu.async_copy(src_tab.at[srcv.at[j1]], rb1, gsem1)
                _scale(rb0, j0)
                pltpu.sync_copy(rb0, acc.at[dstv.at[j0]], add=True)
                pltpu.make_async_copy(src_tab.at[pl.ds(0, 128)], rb1,
                                      gsem1).wait()

                @pl.when(jj + 1 < NPAIR)
                def _():
                    pltpu.async_copy(src_tab.at[srcv.at[j0 + 2]], rb0, gsem0)

                _scale(rb1, j1)
                pltpu.sync_copy(rb1, acc.at[dstv.at[j1]], add=True)
                return 0
            lax.fori_loop(0, NPAIR, _pair, 0)
            plsc.subcore_barrier()

            base = s * NPT
            obase = cN + s * NPT
            for blk in range(NPT // 128):
                pltpu.sync_copy(acc.at[pl.ds(base + blk * 128, 128)], rb0)
                if prev_tab is not None:
                    pltpu.sync_copy(prev_tab.at[pl.ds(obase + blk * 128,
                                                      128)], rb1)

                def _cg(g, _):
                    d16 = disb[pl.ds(blk * 128 + g * 16, 16)]
                    for i in range(16):
                        dr = d16[i]
                        r = g * 16 + i
                        for q in range(NQ):
                            sl = pl.ds(q * 16, 16)
                            if prev_tab is not None:
                                t = (-2.0 * dr) * rb0[r, sl] - rb1[r, sl]
                            else:
                                t = (-dr) * rb0[r, sl]
                            rb0[r, sl] = t
                            if uout_tab is not None:
                                rb1[r, sl] = dr * t
                    return 0
                lax.fori_loop(0, 8, _cg, 0)
                pltpu.sync_copy(rb0, out_tab.at[pl.ds(obase + blk * 128,
                                                      128)])
                if uout_tab is not None:
                    pltpu.sync_copy(rb1, uout_tab.at[pl.ds(obase + blk * 128,
                                                           128)])
            if uout_tab is not None:
                _zero_rb(rb0)
                for blk in range(NPT // 128):
                    pltpu.sync_copy(rb0, acc.at[pl.ds(base + blk * 128,
                                                      128)])
            plsc.subcore_barrier()

        _hop(u0, None, t1, u1o)
        _hop(u1o, x2, t2, u2o)
        _hop(u2o, t1, t3, u3o)
        _hop(u3o, t2, t4, None)

    return _series_sc_body


def _make_series_call(W):
    return pl.kernel(
        _make_series_body(W),
        out_type=[jax.ShapeDtypeStruct((2 * NROW, W), jnp.float32)] * 7,
        mesh=_sc_mesh,
        scratch_types=[
            pltpu.VMEM_SHARED((NROW, W), jnp.float32),
            pltpu.VMEM((NCH, 128), jnp.int32),
            pltpu.VMEM((NCH, 128), jnp.int32),
            pltpu.VMEM((NCH, 128), jnp.float32),
            pltpu.VMEM((128, W), jnp.float32),
            pltpu.VMEM((128, W), jnp.float32),
            pltpu.SemaphoreType.DMA,
            pltpu.SemaphoreType.DMA,
            pltpu.VMEM((NPT,), jnp.float32),
        ],
        compiler_params=pltpu.CompilerParams(use_tc_tiling_on_sc=False),
    )


_series64_call = _make_series_call(64)
_series16_call = _make_series_call(16)


def kernel(edge_index, edge_weight, emb, weights, biases):
    src, dst = edge_index[0], edge_index[1]

    npad = EPAD - E
    esrc = jnp.concatenate([src, jnp.zeros((npad,), jnp.int32)]) \
        .reshape(16, NCH, 128)
    edst = jnp.concatenate(
        [dst, N + (jnp.arange(npad, dtype=jnp.int32) % 16)]) \
        .reshape(16, NCH, 128)
    ew = jnp.concatenate([edge_weight, jnp.zeros((npad,), jnp.float32)]) \
        .reshape(16, NCH, 128)

    deg = jax.ops.segment_sum(edge_weight, src, num_segments=N)
    safe_deg = jnp.where(deg > 0, deg, 1.0)
    dis = jnp.where(deg > 0, lax.rsqrt(safe_deg), 0.0)
    dispad = jnp.pad(dis, (0, NROW - N))

    ind_t = jnp.zeros((2 * NROW, 16), jnp.float32).at[:N, 0].set(1.0)
    ind_u = jnp.zeros((2 * NROW, 16), jnp.float32).at[:N, 0].set(dis)
    tt = _series16_call(ind_t, ind_u, esrc, edst, ew, dispad)
    tpad = jnp.concatenate(
        [jnp.ones((N, 1), jnp.float32)]
        + [t[:N, :1] for t in tt[:4]]
        + [jnp.zeros((N, 3), jnp.float32)], axis=1)

    def series_all(xsplit):
        x2 = jnp.pad(xsplit, ((0, 0), (0, NROW - N), (0, 0))) \
            .reshape(2 * NROW, 64)
        u0 = jnp.pad(xsplit * dis[None, :, None],
                     ((0, 0), (0, NROW - N), (0, 0))).reshape(2 * NROW, 64)
        res = _series64_call(x2, u0, esrc, edst, ew, dispad)
        return jnp.stack(
            [xsplit] + [t.reshape(2, NROW, 64)[:, :N, :] for t in res[:4]])

    cur = jnp.stack([emb[:, :64], emb[:, 64:]])
    hs = []
    for l in range(L):
        W = weights[l]
        b = biases[l]
        wcat = jnp.concatenate([W[0], W[2], W[4]], axis=-1)
        bz = (b[0] + b[1]).reshape(1, H)
        br = (b[2] + b[3]).reshape(1, H)
        bh4 = b[4].reshape(1, H)
        bh5 = b[5].reshape(1, H)

        tx_all = series_all(cur)
        z, rsplit, ahx = _gates_call(tx_all, tpad, wcat, W[1], W[3],
                                     bz, br, bh4)
        tr_all = series_all(rsplit)
        h, hsplit = _out_call(tr_all, W[5], ahx, z, bh5)
        hs.append(h)
        cur = hsplit
    return (hs[-1], hs[0], hs[1])
